# baseline (speedup 1.0000x reference)
import jax
import jax.numpy as jnp
from jax.experimental import pallas as pl
from jax.experimental.pallas import tpu as pltpu

_EPS = 1e-5
_BM = 400


def _fused_kernel(a_ref, x_ref, w_ref, b_ref, g_ref, bt_ref, o_ref):
    h = jnp.dot(a_ref[...], x_ref[...].astype(jnp.float32),
                preferred_element_type=jnp.float32)
    y = jnp.dot(h, w_ref[...], preferred_element_type=jnp.float32) + b_ref[...]
    mu = jnp.mean(y, axis=-1, keepdims=True)
    d = y - mu
    var = jnp.mean(d * d, axis=-1, keepdims=True)
    yn = d * jax.lax.rsqrt(var + _EPS) * g_ref[...] + bt_ref[...]
    o_ref[...] = jnp.maximum(yn, 0.0)


def kernel(A_hat, X, W, b, gamma, beta):
    n, k = A_hat.shape
    d_in = X.shape[1]
    d_out = W.shape[0]
    bm = _BM

    wt = W.T
    b2 = b.reshape(1, d_out)
    g2 = gamma.reshape(1, d_out)
    bt2 = beta.reshape(1, d_out)

    x16 = X.astype(jnp.bfloat16)
    return pl.pallas_call(
        _fused_kernel,
        grid=(pl.cdiv(n, bm),),
        in_specs=[
            pl.BlockSpec((bm, k), lambda i: (i, 0)),
            pl.BlockSpec((k, d_in), lambda i: (0, 0)),
            pl.BlockSpec((d_in, d_out), lambda i: (0, 0)),
            pl.BlockSpec((1, d_out), lambda i: (0, 0)),
            pl.BlockSpec((1, d_out), lambda i: (0, 0)),
            pl.BlockSpec((1, d_out), lambda i: (0, 0)),
        ],
        out_specs=pl.BlockSpec((bm, d_out), lambda i: (i, 0)),
        out_shape=jax.ShapeDtypeStruct((n, d_out), jnp.float32),
        compiler_params=pltpu.CompilerParams(
            dimension_semantics=("parallel",),
        ),
    )(A_hat, x16, wt, b2, g2, bt2)


# speedup vs baseline: 1.0368x; 1.0368x over previous
import jax
import jax.numpy as jnp
from jax.experimental import pallas as pl
from jax.experimental.pallas import tpu as pltpu

_EPS = 1e-5
_BM = 400


def _fused_kernel(a_ref, x_ref, w_ref, b_ref, g_ref, bt_ref, o_ref):
    h = jnp.dot(a_ref[...], x_ref[...], preferred_element_type=jnp.float32)
    y = jax.lax.dot_general(
        h, w_ref[...], (((1,), (1,)), ((), ())),
        preferred_element_type=jnp.float32) + b_ref[...]
    mu = jnp.mean(y, axis=-1, keepdims=True)
    d = y - mu
    var = jnp.mean(d * d, axis=-1, keepdims=True)
    yn = d * jax.lax.rsqrt(var + _EPS) * g_ref[...] + bt_ref[...]
    o_ref[...] = jnp.maximum(yn, 0.0)


def kernel(A_hat, X, W, b, gamma, beta):
    n, k = A_hat.shape
    d_in = X.shape[1]
    d_out = W.shape[0]
    bm = _BM

    b2 = b.reshape(1, d_out)
    g2 = gamma.reshape(1, d_out)
    bt2 = beta.reshape(1, d_out)

    return pl.pallas_call(
        _fused_kernel,
        grid=(pl.cdiv(n, bm),),
        in_specs=[
            pl.BlockSpec((bm, k), lambda i: (i, 0)),
            pl.BlockSpec((k, d_in), lambda i: (0, 0)),
            pl.BlockSpec((d_out, d_in), lambda i: (0, 0)),
            pl.BlockSpec((1, d_out), lambda i: (0, 0)),
            pl.BlockSpec((1, d_out), lambda i: (0, 0)),
            pl.BlockSpec((1, d_out), lambda i: (0, 0)),
        ],
        out_specs=pl.BlockSpec((bm, d_out), lambda i: (i, 0)),
        out_shape=jax.ShapeDtypeStruct((n, d_out), jnp.float32),
        compiler_params=pltpu.CompilerParams(
            dimension_semantics=("parallel",),
        ),
    )(A_hat, X, W, b2, g2, bt2)
